# baseline (speedup 1.0000x reference)
import functools

import jax
import jax.numpy as jnp
from jax import lax
from jax.experimental import pallas as pl
from jax.experimental.pallas import tpu as pltpu
from jax.experimental.pallas import tpu_sc as plsc

D = 64
BATCH = 16384
NC = 2
NS = 16
NW = NC * NS
B_PER_W = BATCH // NW
CHUNK = 128
NCHUNK = B_PER_W // CHUNK


def _sc_gather_pair(uid2, iid2, ut, it):
    mesh = plsc.VectorSubcoreMesh(core_axis_name="c", subcore_axis_name="s")
    row_t = jax.ShapeDtypeStruct((BATCH, D), jnp.float32)

    @functools.partial(
        pl.kernel,
        mesh=mesh,
        out_type=[row_t, row_t],
        compiler_params=pltpu.CompilerParams(use_tc_tiling_on_sc=False),
        scratch_types=[
            pltpu.VMEM((NCHUNK, CHUNK), jnp.int32),
            pltpu.VMEM((NCHUNK, CHUNK), jnp.int32),
            pltpu.VMEM((CHUNK, D), jnp.float32),
            pltpu.VMEM((CHUNK, D), jnp.float32),
            pltpu.SemaphoreType.DMA,
        ],
    )
    def k(uid_hbm, iid_hbm, u_hbm, i_hbm, o_u, o_i, uv, iv, bu, bi, sem):
        wid = lax.axis_index("s") * NC + lax.axis_index("c")
        pltpu.sync_copy(uid_hbm.at[wid], uv)
        pltpu.sync_copy(iid_hbm.at[wid], iv)
        for j in range(NCHUNK):
            cps = [
                pltpu.async_copy(u_hbm.at[uv.at[j]], bu, sem),
                pltpu.async_copy(i_hbm.at[iv.at[j]], bi, sem),
            ]
            for c in cps:
                c.wait()
            base = wid * B_PER_W + j * CHUNK
            pltpu.sync_copy(bu, o_u.at[pl.ds(base, CHUNK)])
            pltpu.sync_copy(bi, o_i.at[pl.ds(base, CHUNK)])

    return k(uid2, iid2, ut, it)


def _mlp_body(umf_ref, imf_ref, umlp_ref, imlp_ref, w1_ref, b1_ref, w2_ref,
              b2_ref, wf1_ref, bf1_ref, wf2r_ref, bf2_ref, out_ref):
    x = jnp.concatenate([umlp_ref[...], imlp_ref[...]], axis=1)
    h = jnp.maximum(
        jnp.dot(x, w1_ref[...], preferred_element_type=jnp.float32)
        + b1_ref[...], 0.0)
    h = jnp.maximum(
        jnp.dot(h, w2_ref[...], preferred_element_type=jnp.float32)
        + b2_ref[...], 0.0)
    mf = umf_ref[...] * imf_ref[...]
    c = jnp.concatenate([mf, h], axis=1)
    o = jnp.maximum(
        jnp.dot(c, wf1_ref[...], preferred_element_type=jnp.float32)
        + bf1_ref[...], 0.0)
    out_ref[...] = jnp.sum(o * wf2r_ref[...], axis=1) + bf2_ref[0, 0]


def _tc_mlp(umf, imf, umlp, imlp, W1, b1, W2, b2, Wf1, bf1, Wf2, bf2,
            interpret=False):
    BM = 2048
    grid = (BATCH // BM,)
    full = lambda r, c: pl.BlockSpec((r, c), lambda m: (0, 0))
    return pl.pallas_call(
        _mlp_body,
        grid=grid,
        in_specs=[
            pl.BlockSpec((BM, D), lambda m: (m, 0)),
            pl.BlockSpec((BM, D), lambda m: (m, 0)),
            pl.BlockSpec((BM, D), lambda m: (m, 0)),
            pl.BlockSpec((BM, D), lambda m: (m, 0)),
            full(2 * D, 128), full(1, 128),
            full(128, D), full(1, D),
            full(2 * D, 32), full(1, 32),
            full(1, 32), full(1, 1),
        ],
        out_specs=pl.BlockSpec((BM,), lambda m: (m,)),
        out_shape=jax.ShapeDtypeStruct((BATCH,), jnp.float32),
        interpret=interpret,
    )(umf, imf, umlp, imlp,
      W1, b1.reshape(1, 128), W2, b2.reshape(1, D),
      Wf1, bf1.reshape(1, 32), Wf2.reshape(1, 32), bf2.reshape(1, 1))


def kernel(user_ids, item_ids, U_mf, I_mf, U_mlp, I_mlp,
           W1, b1, W2, b2, Wf1, bf1, Wf2, bf2):
    uid2 = user_ids.astype(jnp.int32).reshape(NW, NCHUNK, CHUNK)
    iid2 = item_ids.astype(jnp.int32).reshape(NW, NCHUNK, CHUNK)
    umlp, imlp = _sc_gather_pair(uid2, iid2, U_mlp, I_mlp)
    umf, imf = _sc_gather_pair(uid2, iid2, U_mf, I_mf)
    return _tc_mlp(umf, imf, umlp, imlp, W1, b1, W2, b2, Wf1, bf1, Wf2, bf2)


# speedup vs baseline: 1.5044x; 1.5044x over previous
import functools

import jax
import jax.numpy as jnp
from jax import lax
from jax.experimental import pallas as pl
from jax.experimental.pallas import tpu as pltpu
from jax.experimental.pallas import tpu_sc as plsc

D = 64
BATCH = 16384
NC = 2
NS = 16
NW = NC * NS
B_PER_W = BATCH // NW


def _sc_gather(uid, iid, U_mf, I_mf, U_mlp, I_mlp):
    mesh = plsc.VectorSubcoreMesh(core_axis_name="c", subcore_axis_name="s")
    out_t = jax.ShapeDtypeStruct((BATCH // 8, 8, 128), jnp.float32)

    @functools.partial(
        pl.kernel,
        mesh=mesh,
        out_type=[out_t, out_t, out_t, out_t],
        scratch_types=[
            pltpu.VMEM((B_PER_W,), jnp.int32),
            pltpu.VMEM((B_PER_W,), jnp.int32),
            pltpu.VMEM((16, 8, 128), jnp.float32),
            pltpu.VMEM((16, 8, 128), jnp.float32),
            pltpu.VMEM((16, 8, 128), jnp.float32),
            pltpu.VMEM((16, 8, 128), jnp.float32),
            pltpu.SemaphoreType.DMA,
            pltpu.SemaphoreType.DMA,
            pltpu.SemaphoreType.DMA,
            pltpu.SemaphoreType.DMA,
        ],
    )
    def k(uid_hbm, iid_hbm, umf, imf, umlp, imlp,
          o_umf, o_imf, o_umlp, o_imlp,
          uid_v, iid_v, b0, b1, b2, b3, sem0, sem1, sem2, sem3):
        wid = lax.axis_index("s") * NC + lax.axis_index("c")
        base = wid * B_PER_W
        pltpu.sync_copy(uid_hbm.at[pl.ds(base, B_PER_W)], uid_v)
        pltpu.sync_copy(iid_hbm.at[pl.ds(base, B_PER_W)], iid_v)

        for c4 in range(4):
            def body(g, carry, c4=c4):
                vu = uid_v[pl.ds(c4 * 128 + g * 16, 16)]
                vi = iid_v[pl.ds(c4 * 128 + g * 16, 16)]
                cps = []
                for l in range(16):
                    jj = g * 16 + l
                    dst = (jj // 8, jj % 8, pl.ds(0, D))
                    cps.append(pltpu.async_copy(
                        umf.at[vu[l]], b0.at[dst[0], dst[1], dst[2]], sem0))
                    cps.append(pltpu.async_copy(
                        imf.at[vi[l]], b1.at[dst[0], dst[1], dst[2]], sem1))
                    cps.append(pltpu.async_copy(
                        umlp.at[vu[l]], b2.at[dst[0], dst[1], dst[2]], sem2))
                    cps.append(pltpu.async_copy(
                        imlp.at[vi[l]], b3.at[dst[0], dst[1], dst[2]], sem3))
                for c in cps:
                    c.wait()
                return carry

            lax.fori_loop(0, 8, body, 0)
            s0 = (base + c4 * 128) // 8
            pltpu.sync_copy(b0, o_umf.at[pl.ds(s0, 16)])
            pltpu.sync_copy(b1, o_imf.at[pl.ds(s0, 16)])
            pltpu.sync_copy(b2, o_umlp.at[pl.ds(s0, 16)])
            pltpu.sync_copy(b3, o_imlp.at[pl.ds(s0, 16)])

    return k(uid, iid, U_mf, I_mf, U_mlp, I_mlp)


def _mlp_body(umf_ref, imf_ref, umlp_ref, imlp_ref, w1_ref, b1_ref, w2_ref,
              b2_ref, wf1_ref, bf1_ref, wf2r_ref, bf2_ref, out_ref):
    x = jnp.concatenate([umlp_ref[:, :D], imlp_ref[:, :D]], axis=1)
    h = jnp.maximum(
        jnp.dot(x, w1_ref[...], preferred_element_type=jnp.float32)
        + b1_ref[...], 0.0)
    h = jnp.maximum(
        jnp.dot(h, w2_ref[...], preferred_element_type=jnp.float32)
        + b2_ref[...], 0.0)
    mf = umf_ref[:, :D] * imf_ref[:, :D]
    c = jnp.concatenate([mf, h], axis=1)
    o = jnp.maximum(
        jnp.dot(c, wf1_ref[...], preferred_element_type=jnp.float32)
        + bf1_ref[...], 0.0)
    out_ref[...] = jnp.sum(o * wf2r_ref[...], axis=1) + bf2_ref[0, 0]


def _tc_mlp(umf, imf, umlp, imlp, W1, b1, W2, b2, Wf1, bf1, Wf2, bf2,
            interpret=False):
    BM = 2048
    grid = (BATCH // BM,)
    full = lambda r, c: pl.BlockSpec((r, c), lambda m: (0, 0))
    return pl.pallas_call(
        _mlp_body,
        grid=grid,
        in_specs=[
            pl.BlockSpec((BM, 128), lambda m: (m, 0)),
            pl.BlockSpec((BM, 128), lambda m: (m, 0)),
            pl.BlockSpec((BM, 128), lambda m: (m, 0)),
            pl.BlockSpec((BM, 128), lambda m: (m, 0)),
            full(2 * D, 128), full(1, 128),
            full(128, D), full(1, D),
            full(2 * D, 32), full(1, 32),
            full(1, 32), full(1, 1),
        ],
        out_specs=pl.BlockSpec((BM,), lambda m: (m,)),
        out_shape=jax.ShapeDtypeStruct((BATCH,), jnp.float32),
        interpret=interpret,
    )(umf, imf, umlp, imlp,
      W1, b1.reshape(1, 128), W2, b2.reshape(1, D),
      Wf1, bf1.reshape(1, 32), Wf2.reshape(1, 32), bf2.reshape(1, 1))


def kernel(user_ids, item_ids, U_mf, I_mf, U_mlp, I_mlp,
           W1, b1, W2, b2, Wf1, bf1, Wf2, bf2):
    uid = user_ids.astype(jnp.int32)
    iid = item_ids.astype(jnp.int32)
    outs = _sc_gather(uid, iid, U_mf, I_mf, U_mlp, I_mlp)
    umf, imf, umlp, imlp = (o.reshape(BATCH, 128) for o in outs)
    return _tc_mlp(umf, imf, umlp, imlp, W1, b1, W2, b2, Wf1, bf1, Wf2, bf2)
